# baseline (speedup 1.0000x reference)
import jax
import jax.numpy as jnp
from jax.experimental import pallas as pl

_BETA = 0.25
_BM = 512


def _rvq_body(x_ref, cb_ref, cbt_ref, e2_ref, xq_ref, idx_ref, loss_ref):
    levels, n_codes, _ = cb_ref.shape
    xb = x_ref[...]
    r = xb
    xq = jnp.zeros_like(xb)
    loss_acc = jnp.zeros((), jnp.float32)
    iota = jax.lax.broadcasted_iota(jnp.int32, (xb.shape[0], n_codes), 1)
    idx_cols = []
    for i in range(levels):
        e = cb_ref[i]
        et = cbt_ref[i]
        z2 = jnp.sum(r * r, axis=1, keepdims=True)
        m = jax.lax.dot_general(
            r, et, (((1,), (0,)), ((), ())),
            preferred_element_type=jnp.float32,
        )
        d = z2 + e2_ref[i][None, :] - 2.0 * m
        minv = jnp.min(d, axis=1, keepdims=True)
        idxv = jnp.min(jnp.where(d == minv, iota, n_codes), axis=1, keepdims=True)
        onehot = (iota == idxv).astype(jnp.float32)
        zq = jax.lax.dot_general(
            onehot, e, (((1,), (0,)), ((), ())),
            preferred_element_type=jnp.float32,
            precision=jax.lax.Precision.HIGHEST,
        )
        loss_acc = loss_acc + jnp.sum((zq - r) ** 2)
        zq_st = r + (zq - r)
        xq = xq + zq_st
        r = r - zq_st
        idx_cols.append(idxv)
    xq_ref[...] = xq
    idx_ref[...] = jnp.concatenate(idx_cols, axis=1)

    @pl.when(pl.program_id(0) == 0)
    def _init():
        loss_ref[...] = jnp.zeros_like(loss_ref)

    loss_ref[...] += jnp.broadcast_to(loss_acc, loss_ref.shape)


def kernel(x, codebooks):
    batch, dim = x.shape
    levels, n_codes, _ = codebooks.shape
    cbt = jnp.transpose(codebooks, (0, 2, 1))
    e2 = jnp.sum(codebooks * codebooks, axis=2)
    nb = batch // _BM
    x_q, idx, loss_buf = pl.pallas_call(
        _rvq_body,
        grid=(nb,),
        in_specs=[
            pl.BlockSpec((_BM, dim), lambda i: (i, 0)),
            pl.BlockSpec((levels, n_codes, dim), lambda i: (0, 0, 0)),
            pl.BlockSpec((levels, dim, n_codes), lambda i: (0, 0, 0)),
            pl.BlockSpec((levels, n_codes), lambda i: (0, 0)),
        ],
        out_specs=[
            pl.BlockSpec((_BM, dim), lambda i: (i, 0)),
            pl.BlockSpec((_BM, levels), lambda i: (i, 0)),
            pl.BlockSpec((1, 128), lambda i: (0, 0)),
        ],
        out_shape=[
            jax.ShapeDtypeStruct((batch, dim), jnp.float32),
            jax.ShapeDtypeStruct((batch, levels), jnp.int32),
            jax.ShapeDtypeStruct((1, 128), jnp.float32),
        ],
    )(x, codebooks, cbt, e2)
    mean_loss = (1.0 + _BETA) * loss_buf[0, 0] / (levels * batch * dim)
    return x_q, mean_loss, idx


# speedup vs baseline: 1.8445x; 1.8445x over previous
import jax
import jax.numpy as jnp
from jax.experimental import pallas as pl

_BETA = 0.25
_BM = 512


def _rvq_body(x_ref, ehi_ref, elo_ref, cbt2_ref, e2_ref, xq_ref, idx_ref,
              loss_ref):
    levels, n_codes, _ = ehi_ref.shape
    xb = x_ref[...]
    r = xb
    xq = jnp.zeros_like(xb)
    loss_acc = jnp.zeros((), jnp.float32)
    iota = jax.lax.broadcasted_iota(jnp.int32, (xb.shape[0], n_codes), 1)
    idx_cols = []
    for i in range(levels):
        z2 = jnp.sum(r * r, axis=1, keepdims=True)
        m2 = jax.lax.dot_general(
            r, cbt2_ref[i], (((1,), (0,)), ((), ())),
            preferred_element_type=jnp.float32,
        )
        d = (z2 + e2_ref[i][None, :]) + m2
        minv = jnp.min(d, axis=1, keepdims=True)
        idxv = jnp.min(jnp.where(d == minv, iota, n_codes), axis=1, keepdims=True)
        onehot = (iota == idxv).astype(jnp.float32).astype(jnp.bfloat16)
        zq = (
            jax.lax.dot_general(
                onehot, ehi_ref[i], (((1,), (0,)), ((), ())),
                preferred_element_type=jnp.float32,
            )
            + jax.lax.dot_general(
                onehot, elo_ref[i], (((1,), (0,)), ((), ())),
                preferred_element_type=jnp.float32,
            )
        )
        loss_acc = loss_acc + jnp.sum((zq - r) ** 2)
        zq_st = r + (zq - r)
        xq = xq + zq_st
        r = r - zq_st
        idx_cols.append(idxv)
    xq_ref[...] = xq
    idx_ref[...] = jnp.concatenate(idx_cols, axis=1)

    @pl.when(pl.program_id(0) == 0)
    def _init():
        loss_ref[...] = jnp.zeros_like(loss_ref)

    loss_ref[...] += jnp.broadcast_to(loss_acc, loss_ref.shape)


def kernel(x, codebooks):
    batch, dim = x.shape
    levels, n_codes, _ = codebooks.shape
    cbt2 = jnp.transpose(-2.0 * codebooks, (0, 2, 1))
    e_hi = codebooks.astype(jnp.bfloat16)
    e_lo = (codebooks - e_hi.astype(jnp.float32)).astype(jnp.bfloat16)
    e2 = jnp.sum(codebooks * codebooks, axis=2)
    nb = batch // _BM
    x_q, idx, loss_buf = pl.pallas_call(
        _rvq_body,
        grid=(nb,),
        in_specs=[
            pl.BlockSpec((_BM, dim), lambda i: (i, 0)),
            pl.BlockSpec((levels, n_codes, dim), lambda i: (0, 0, 0)),
            pl.BlockSpec((levels, n_codes, dim), lambda i: (0, 0, 0)),
            pl.BlockSpec((levels, dim, n_codes), lambda i: (0, 0, 0)),
            pl.BlockSpec((levels, n_codes), lambda i: (0, 0)),
        ],
        out_specs=[
            pl.BlockSpec((_BM, dim), lambda i: (i, 0)),
            pl.BlockSpec((_BM, levels), lambda i: (i, 0)),
            pl.BlockSpec((1, 128), lambda i: (0, 0)),
        ],
        out_shape=[
            jax.ShapeDtypeStruct((batch, dim), jnp.float32),
            jax.ShapeDtypeStruct((batch, levels), jnp.int32),
            jax.ShapeDtypeStruct((1, 128), jnp.float32),
        ],
    )(x, e_hi, e_lo, cbt2, e2)
    mean_loss = (1.0 + _BETA) * loss_buf[0, 0] / (levels * batch * dim)
    return x_q, mean_loss, idx
